# Initial kernel scaffold; baseline (speedup 1.0000x reference)
import jax
import jax.numpy as jnp
from jax.experimental import pallas as pl


def kernel(x, edge_index, edge_attr, batch, emb, We1, be1, W1a, b1a, W1b, b1b, We2, be2, W2a, b2a, W2b, b2b, g1, beta1, g2, beta2, Wv1, bv1, Wv2, bv2, Wl, bl):
    raise NotImplementedError("write your pallas kernel here")


# speedup vs baseline: 2.5104x; 2.5104x over previous
import functools

import jax
import jax.numpy as jnp
from jax.experimental import pallas as pl
from jax.experimental.pallas import tpu as pltpu
from jax.experimental.pallas import tpu_sc as plsc

N = 10000
E = 320000
H = 128
G = 64

NSC = 2
NSUB = 16
SCBLK = 128
SCBLK2 = 128
EPAD = ((E + NSC * NSUB * SCBLK - 1) // (NSC * NSUB * SCBLK)) * (NSC * NSUB * SCBLK)
NPAD = 10240
ROWS_PER_SUB = NPAD // NSUB

BE = 2048


def _vector_mesh():
    return plsc.VectorSubcoreMesh(core_axis_name="core", subcore_axis_name="subcore")


def _edge1_body(ea_ref, w_ref, c_ref, o_ref):
    i = pl.program_id(0)
    e = jnp.dot(ea_ref[...], w_ref[...], preferred_element_type=jnp.float32)
    v = jnp.maximum(e + c_ref[...], 0.0)
    rows = i * BE + jax.lax.broadcasted_iota(jnp.int32, (BE, H), 0)
    o_ref[...] = jnp.where(rows < E, v, 0.0)


def _edge1(ea_p, w, c):
    return pl.pallas_call(
        _edge1_body,
        grid=(EPAD // BE,),
        in_specs=[
            pl.BlockSpec((BE, 8), lambda i: (i, 0)),
            pl.BlockSpec((8, H), lambda i: (0, 0)),
            pl.BlockSpec((1, H), lambda i: (0, 0)),
        ],
        out_specs=pl.BlockSpec((BE, H), lambda i: (i, 0)),
        out_shape=jax.ShapeDtypeStruct((EPAD, H), jnp.float32),
    )(ea_p, w, c)


def _edge2_body(ea_ref, w_ref, b_ref, o_ref):
    i = pl.program_id(0)
    e = jnp.dot(ea_ref[...], w_ref[...], preferred_element_type=jnp.float32)
    v = e + b_ref[...]
    rows = i * BE + jax.lax.broadcasted_iota(jnp.int32, (BE, H), 0)
    o_ref[...] = jnp.where(rows < E, v, -1e30)


def _edge2(ea_p, w, b):
    return pl.pallas_call(
        _edge2_body,
        grid=(EPAD // BE,),
        in_specs=[
            pl.BlockSpec((BE, 8), lambda i: (i, 0)),
            pl.BlockSpec((8, H), lambda i: (0, 0)),
            pl.BlockSpec((1, H), lambda i: (0, 0)),
        ],
        out_specs=pl.BlockSpec((BE, H), lambda i: (i, 0)),
        out_shape=jax.ShapeDtypeStruct((EPAD, H), jnp.float32),
    )(ea_p, w, b)


def _node_body(p_ref, xp_ref, wa_ref, ba_ref, wb_ref, bb_ref, g_ref, beta_ref,
               o_ref):
    hin = xp_ref[...] + p_ref[0][:N] + p_ref[1][:N]
    t = jnp.maximum(
        jnp.dot(hin, wa_ref[...], preferred_element_type=jnp.float32)
        + ba_ref[...], 0.0)
    t = jnp.dot(t, wb_ref[...], preferred_element_type=jnp.float32) + bb_ref[...]
    mu = jnp.mean(t, axis=0, keepdims=True)
    var = jnp.mean(jnp.square(t - mu), axis=0, keepdims=True)
    y = g_ref[...] * (t - mu) * jax.lax.rsqrt(var + 1e-5) + beta_ref[...]
    o_ref[...] = jnp.maximum(y + xp_ref[...], 0.0)


def _node(p, xp, wa, ba, wb, bb, g, beta):
    return pl.pallas_call(
        _node_body,
        out_shape=jax.ShapeDtypeStruct((N, H), jnp.float32),
    )(p, xp, wa, ba, wb, bb, g, beta)


def _final_body(p_ref, h1_ref, wa_ref, ba_ref, wb_ref, bb_ref, g_ref, beta_ref,
                b_ref, wv1_ref, bv1_ref, wv2_ref, bv2_ref, wl_ref, bl_ref,
                out_ref, h_ref):
    hin = h1_ref[...] + p_ref[0][:N] + p_ref[1][:N]
    t = jnp.maximum(
        jnp.dot(hin, wa_ref[...], preferred_element_type=jnp.float32)
        + ba_ref[...], 0.0)
    t = jnp.dot(t, wb_ref[...], preferred_element_type=jnp.float32) + bb_ref[...]
    mu = jnp.mean(t, axis=0, keepdims=True)
    var = jnp.mean(jnp.square(t - mu), axis=0, keepdims=True)
    y = g_ref[...] * (t - mu) * jax.lax.rsqrt(var + 1e-5) + beta_ref[...]
    h2 = jnp.maximum(y + h1_ref[...], 0.0)

    onehot = (jax.lax.broadcasted_iota(jnp.int32, (G, N), 0)
              == b_ref[...]).astype(jnp.float32)
    cnt = jnp.sum(onehot, axis=1, keepdims=True)
    inv = 1.0 / jnp.maximum(cnt, 1.0)
    gm = jnp.dot(onehot, h2, preferred_element_type=jnp.float32) * inv
    gp = jnp.maximum(
        jnp.dot(gm, wv1_ref[...], preferred_element_type=jnp.float32)
        + bv1_ref[...], 0.0)
    gp = jnp.dot(gp, wv2_ref[...], preferred_element_type=jnp.float32) + bv2_ref[...]
    gexp = jax.lax.dot_general(onehot, gp, (((0,), (0,)), ((), ())),
                               preferred_element_type=jnp.float32)
    h = h2 + gexp
    gr = jnp.dot(onehot, h, preferred_element_type=jnp.float32) * inv
    out_ref[...] = jnp.dot(gr, wl_ref[...], preferred_element_type=jnp.float32) \
        + bl_ref[...]
    h_ref[...] = h


def _final(p, h1, wa, ba, wb, bb, g, beta, batch2, wv1, bv1, wv2, bv2, wl, bl):
    return pl.pallas_call(
        _final_body,
        out_shape=[
            jax.ShapeDtypeStruct((G, H), jnp.float32),
            jax.ShapeDtypeStruct((N, H), jnp.float32),
        ],
    )(p, h1, wa, ba, wb, bb, g, beta, batch2, wv1, bv1, wv2, bv2, wl, bl)


def _scatter_add(m, dst2, zeros_n):
    @functools.partial(
        pl.kernel,
        out_type=jax.ShapeDtypeStruct((NSC, NPAD, H), jnp.float32),
        mesh=_vector_mesh(),
        scratch_types=[pltpu.VMEM_SHARED((NPAD, H), jnp.float32)],
    )
    def k(m_hbm, dst_hbm, z_hbm, out_hbm, acc_sh):
        c = jax.lax.axis_index("core")
        s = jax.lax.axis_index("subcore")
        r0 = s * ROWS_PER_SUB
        pltpu.sync_copy(z_hbm.at[pl.ds(r0, ROWS_PER_SUB)],
                        acc_sh.at[pl.ds(r0, ROWS_PER_SUB)])
        plsc.subcore_barrier()

        def body(m_vmem, i_vmem):
            pltpu.sync_copy(m_vmem, acc_sh.at[i_vmem.at[0]], add=True)

        pltpu.emit_pipeline(
            body,
            grid=(EPAD // SCBLK,),
            in_specs=[
                pl.BlockSpec((SCBLK, H), lambda i: (i, 0)),
                pl.BlockSpec((1, SCBLK), lambda i: (0, i)),
            ],
            core_axis_name=("core", "subcore"),
            dimension_semantics=(pltpu.PARALLEL,),
        )(m_hbm, dst_hbm)

        plsc.subcore_barrier()
        pltpu.sync_copy(acc_sh.at[pl.ds(r0, ROWS_PER_SUB)],
                        out_hbm.at[c, pl.ds(r0, ROWS_PER_SUB)])

    return k(m, dst2, zeros_n)


def _fused2(e2m, h1, src2, dst2, zeros_n):
    @functools.partial(
        pl.kernel,
        out_type=jax.ShapeDtypeStruct((NSC, NPAD, H), jnp.float32),
        mesh=_vector_mesh(),
        scratch_types=[
            pltpu.VMEM_SHARED((NPAD, H), jnp.float32),
            pltpu.VMEM((SCBLK2 // 2, H), jnp.float32),
        ],
    )
    def k(e_hbm, h_hbm, src_hbm, dst_hbm, z_hbm, out_hbm, acc_sh, hs_buf):
        c = jax.lax.axis_index("core")
        s = jax.lax.axis_index("subcore")
        r0 = s * ROWS_PER_SUB
        HB = SCBLK2 // 2
        pltpu.sync_copy(z_hbm.at[pl.ds(0, HB)], hs_buf)

        @pl.loop(0, ROWS_PER_SUB, step=HB)
        def _(r):
            pltpu.sync_copy(hs_buf, acc_sh.at[pl.ds(r0 + r, HB)])

        plsc.subcore_barrier()

        def body(e_vmem, s_vmem, d_vmem):
            for half in range(2):
                pltpu.sync_copy(
                    h_hbm.at[s_vmem.at[0, pl.ds(half * HB, HB)]], hs_buf)

                @pl.loop(0, HB)
                def _(i):
                    for j in range(H // 16):
                        slo = (pl.ds(half * HB + i, 1), pl.ds(j * 16, 16))
                        sli = (pl.ds(i, 1), pl.ds(j * 16, 16))
                        e_vmem.at[slo][...] = jnp.maximum(
                            e_vmem.at[slo][...] + hs_buf.at[sli][...], 0.0)

            pltpu.sync_copy(e_vmem, acc_sh.at[d_vmem.at[0]], add=True)

        pltpu.emit_pipeline(
            body,
            grid=(EPAD // SCBLK2,),
            in_specs=[
                pl.BlockSpec((SCBLK2, H), lambda i: (i, 0)),
                pl.BlockSpec((1, SCBLK2), lambda i: (0, i)),
                pl.BlockSpec((1, SCBLK2), lambda i: (0, i)),
            ],
            core_axis_name=("core", "subcore"),
            dimension_semantics=(pltpu.PARALLEL,),
        )(e_hbm, src_hbm, dst_hbm)

        plsc.subcore_barrier()

        @pl.loop(0, ROWS_PER_SUB, step=HB)
        def _(r):
            pltpu.sync_copy(acc_sh.at[pl.ds(r0 + r, HB)], hs_buf)
            pltpu.sync_copy(hs_buf, out_hbm.at[c, pl.ds(r0 + r, HB)])

    return k(e2m, h1, src2, dst2, zeros_n)


def kernel(x, edge_index, edge_attr, batch, emb, We1, be1, W1a, b1a, W1b, b1b,
           We2, be2, W2a, b2a, W2b, b2b, g1, beta1, g2, beta2, Wv1, bv1, Wv2,
           bv2, Wl, bl):
    src = edge_index[0]
    dst = edge_index[1]
    pad = EPAD - E
    ea_p = jnp.pad(edge_attr, ((0, pad), (0, 1)))
    src2 = jnp.pad(src, (0, pad)).reshape(1, EPAD)
    dst2 = jnp.pad(dst, (0, pad)).reshape(1, EPAD)
    zeros_n = jnp.zeros((NPAD, H), jnp.float32)
    emb0 = emb[0:1]
    We1p = jnp.pad(We1, ((0, 1), (0, 0)))
    We2p = jnp.pad(We2, ((0, 1), (0, 0)))

    m1 = _edge1(ea_p, We1p, emb0 + be1[None, :])
    p1 = _scatter_add(m1, dst2, zeros_n)
    h1 = _node(p1, emb0, W1a, b1a[None, :], W1b, b1b[None, :],
               g1[None, :], beta1[None, :])
    e2m = _edge2(ea_p, We2p, be2[None, :])
    p2 = _fused2(e2m, h1, src2, dst2, zeros_n)
    out, h = _final(p2, h1, W2a, b2a[None, :], W2b, b2b[None, :],
                    g2[None, :], beta2[None, :], batch.reshape(1, N),
                    Wv1, bv1[None, :], Wv2, bv2[None, :], Wl, bl[None, :])
    return out, h


# speedup vs baseline: 4.2917x; 1.7096x over previous
import functools

import jax
import jax.numpy as jnp
from jax.experimental import pallas as pl
from jax.experimental.pallas import tpu as pltpu
from jax.experimental.pallas import tpu_sc as plsc

N = 10000
E = 320000
H = 128
G = 64

NSC = 2
NSUB = 16
SCBLK = 128
SCBLK2 = 128
EPAD = ((E + NSC * NSUB * SCBLK - 1) // (NSC * NSUB * SCBLK)) * (NSC * NSUB * SCBLK)
NPAD = 10240
ROWS_PER_SUB = NPAD // NSUB

BE = 2048


def _vector_mesh():
    return plsc.VectorSubcoreMesh(core_axis_name="core", subcore_axis_name="subcore")


def _edge1_body(ea_ref, w_ref, c_ref, o_ref):
    i = pl.program_id(0)
    e = jnp.dot(ea_ref[...], w_ref[...], preferred_element_type=jnp.float32)
    v = jnp.maximum(e + c_ref[...], 0.0)
    rows = i * BE + jax.lax.broadcasted_iota(jnp.int32, (BE, H), 0)
    o_ref[...] = jnp.where(rows < E, v, 0.0)


def _edge1(ea_p, w, c):
    return pl.pallas_call(
        _edge1_body,
        grid=(EPAD // BE,),
        in_specs=[
            pl.BlockSpec((BE, 8), lambda i: (i, 0)),
            pl.BlockSpec((8, H), lambda i: (0, 0)),
            pl.BlockSpec((1, H), lambda i: (0, 0)),
        ],
        out_specs=pl.BlockSpec((BE, H), lambda i: (i, 0)),
        out_shape=jax.ShapeDtypeStruct((EPAD, H), jnp.float32),
    )(ea_p, w, c)


def _edge2_body(ea_ref, w_ref, b_ref, hs_ref, o_ref):
    i = pl.program_id(0)
    e = jnp.dot(ea_ref[...], w_ref[...], preferred_element_type=jnp.float32)
    v = jnp.maximum(hs_ref[...] + e + b_ref[...], 0.0)
    rows = i * BE + jax.lax.broadcasted_iota(jnp.int32, (BE, H), 0)
    o_ref[...] = jnp.where(rows < E, v, 0.0)


def _edge2(ea_p, w, b, hsrc):
    return pl.pallas_call(
        _edge2_body,
        grid=(EPAD // BE,),
        in_specs=[
            pl.BlockSpec((BE, 8), lambda i: (i, 0)),
            pl.BlockSpec((8, H), lambda i: (0, 0)),
            pl.BlockSpec((1, H), lambda i: (0, 0)),
            pl.BlockSpec((BE, H), lambda i: (i, 0)),
        ],
        out_specs=pl.BlockSpec((BE, H), lambda i: (i, 0)),
        out_shape=jax.ShapeDtypeStruct((EPAD, H), jnp.float32),
    )(ea_p, w, b, hsrc)


def _node_body(p_ref, xp_ref, wa_ref, ba_ref, wb_ref, bb_ref, g_ref, beta_ref,
               o_ref):
    hin = xp_ref[...] + p_ref[0][:N] + p_ref[1][:N]
    t = jnp.maximum(
        jnp.dot(hin, wa_ref[...], preferred_element_type=jnp.float32)
        + ba_ref[...], 0.0)
    t = jnp.dot(t, wb_ref[...], preferred_element_type=jnp.float32) + bb_ref[...]
    mu = jnp.mean(t, axis=0, keepdims=True)
    var = jnp.mean(jnp.square(t - mu), axis=0, keepdims=True)
    y = g_ref[...] * (t - mu) * jax.lax.rsqrt(var + 1e-5) + beta_ref[...]
    o_ref[...] = jnp.maximum(y + xp_ref[...], 0.0)


def _node(p, xp, wa, ba, wb, bb, g, beta):
    return pl.pallas_call(
        _node_body,
        out_shape=jax.ShapeDtypeStruct((N, H), jnp.float32),
    )(p, xp, wa, ba, wb, bb, g, beta)


def _final_body(p_ref, h1_ref, wa_ref, ba_ref, wb_ref, bb_ref, g_ref, beta_ref,
                b_ref, wv1_ref, bv1_ref, wv2_ref, bv2_ref, wl_ref, bl_ref,
                out_ref, h_ref):
    hin = h1_ref[...] + p_ref[0][:N] + p_ref[1][:N]
    t = jnp.maximum(
        jnp.dot(hin, wa_ref[...], preferred_element_type=jnp.float32)
        + ba_ref[...], 0.0)
    t = jnp.dot(t, wb_ref[...], preferred_element_type=jnp.float32) + bb_ref[...]
    mu = jnp.mean(t, axis=0, keepdims=True)
    var = jnp.mean(jnp.square(t - mu), axis=0, keepdims=True)
    y = g_ref[...] * (t - mu) * jax.lax.rsqrt(var + 1e-5) + beta_ref[...]
    h2 = jnp.maximum(y + h1_ref[...], 0.0)

    onehot = (jax.lax.broadcasted_iota(jnp.int32, (G, N), 0)
              == b_ref[...]).astype(jnp.float32)
    cnt = jnp.sum(onehot, axis=1, keepdims=True)
    inv = 1.0 / jnp.maximum(cnt, 1.0)
    gm = jnp.dot(onehot, h2, preferred_element_type=jnp.float32) * inv
    gp = jnp.maximum(
        jnp.dot(gm, wv1_ref[...], preferred_element_type=jnp.float32)
        + bv1_ref[...], 0.0)
    gp = jnp.dot(gp, wv2_ref[...], preferred_element_type=jnp.float32) + bv2_ref[...]
    gexp = jax.lax.dot_general(onehot, gp, (((0,), (0,)), ((), ())),
                               preferred_element_type=jnp.float32)
    h = h2 + gexp
    gr = jnp.dot(onehot, h, preferred_element_type=jnp.float32) * inv
    out_ref[...] = jnp.dot(gr, wl_ref[...], preferred_element_type=jnp.float32) \
        + bl_ref[...]
    h_ref[...] = h


def _final(p, h1, wa, ba, wb, bb, g, beta, batch2, wv1, bv1, wv2, bv2, wl, bl):
    return pl.pallas_call(
        _final_body,
        out_shape=[
            jax.ShapeDtypeStruct((G, H), jnp.float32),
            jax.ShapeDtypeStruct((N, H), jnp.float32),
        ],
    )(p, h1, wa, ba, wb, bb, g, beta, batch2, wv1, bv1, wv2, bv2, wl, bl)


def _scatter_add(m, dst2, zeros_n):
    @functools.partial(
        pl.kernel,
        out_type=jax.ShapeDtypeStruct((NSC, NPAD, H), jnp.float32),
        mesh=_vector_mesh(),
        scratch_types=[pltpu.VMEM_SHARED((NPAD, H), jnp.float32)],
    )
    def k(m_hbm, dst_hbm, z_hbm, out_hbm, acc_sh):
        c = jax.lax.axis_index("core")
        s = jax.lax.axis_index("subcore")
        r0 = s * ROWS_PER_SUB
        pltpu.sync_copy(z_hbm.at[pl.ds(r0, ROWS_PER_SUB)],
                        acc_sh.at[pl.ds(r0, ROWS_PER_SUB)])
        plsc.subcore_barrier()

        def body(m_vmem, i_vmem):
            pltpu.sync_copy(m_vmem, acc_sh.at[i_vmem.at[0]], add=True)

        pltpu.emit_pipeline(
            body,
            grid=(EPAD // SCBLK,),
            in_specs=[
                pl.BlockSpec((SCBLK, H), lambda i: (i, 0)),
                pl.BlockSpec((1, SCBLK), lambda i: (0, i)),
            ],
            core_axis_name=("core", "subcore"),
            dimension_semantics=(pltpu.PARALLEL,),
        )(m_hbm, dst_hbm)

        plsc.subcore_barrier()
        pltpu.sync_copy(acc_sh.at[pl.ds(r0, ROWS_PER_SUB)],
                        out_hbm.at[c, pl.ds(r0, ROWS_PER_SUB)])

    return k(m, dst2, zeros_n)


def _gather(h1p, src2):
    @functools.partial(
        pl.kernel,
        out_type=jax.ShapeDtypeStruct((EPAD, H), jnp.float32),
        mesh=_vector_mesh(),
        scratch_types=[pltpu.VMEM_SHARED((NPAD, H), jnp.float32)],
    )
    def k(h_hbm, i_hbm, o_hbm, h_sh):
        s_ = jax.lax.axis_index("subcore")
        r0 = s_ * ROWS_PER_SUB

        @pl.loop(0, ROWS_PER_SUB, step=SCBLK)
        def _(r):
            pltpu.sync_copy(h_hbm.at[pl.ds(r0 + r, SCBLK)],
                            h_sh.at[pl.ds(r0 + r, SCBLK)])

        plsc.subcore_barrier()

        def body(i_vmem, o_vmem):
            pltpu.sync_copy(h_sh.at[i_vmem.at[0]], o_vmem)

        pltpu.emit_pipeline(
            body,
            grid=(EPAD // SCBLK,),
            in_specs=[pl.BlockSpec((1, SCBLK), lambda i: (0, i))],
            out_specs=[pl.BlockSpec((SCBLK, H), lambda i: (i, 0))],
            core_axis_name=("core", "subcore"),
            dimension_semantics=(pltpu.PARALLEL,),
        )(i_hbm, o_hbm)

    return k(h1p, src2)


def kernel(x, edge_index, edge_attr, batch, emb, We1, be1, W1a, b1a, W1b, b1b,
           We2, be2, W2a, b2a, W2b, b2b, g1, beta1, g2, beta2, Wv1, bv1, Wv2,
           bv2, Wl, bl):
    src = edge_index[0]
    dst = edge_index[1]
    pad = EPAD - E
    ea_p = jnp.pad(edge_attr, ((0, pad), (0, 1)))
    src2 = jnp.pad(src, (0, pad)).reshape(1, EPAD)
    dst2 = jnp.pad(dst, (0, pad)).reshape(1, EPAD)
    zeros_n = jnp.zeros((NPAD, H), jnp.float32)
    emb0 = emb[0:1]
    We1p = jnp.pad(We1, ((0, 1), (0, 0)))
    We2p = jnp.pad(We2, ((0, 1), (0, 0)))

    m1 = _edge1(ea_p, We1p, emb0 + be1[None, :])
    p1 = _scatter_add(m1, dst2, zeros_n)
    h1 = _node(p1, emb0, W1a, b1a[None, :], W1b, b1b[None, :],
               g1[None, :], beta1[None, :])
    hsrc = _gather(jnp.pad(h1, ((0, NPAD - N), (0, 0))), src2)
    m2 = _edge2(ea_p, We2p, be2[None, :], hsrc)
    p2 = _scatter_add(m2, dst2, zeros_n)
    out, h = _final(p2, h1, W2a, b2a[None, :], W2b, b2b[None, :],
                    g2[None, :], beta2[None, :], batch.reshape(1, N),
                    Wv1, bv1[None, :], Wv2, bv2[None, :], Wl, bl[None, :])
    return out, h
